# baseline (speedup 1.0000x reference)
import functools

import jax
import jax.numpy as jnp
from jax import lax
from jax.experimental import pallas as pl
from jax.experimental.pallas import tpu as pltpu
from jax.experimental.pallas import tpu_sc as plsc

_N = 10000
_E = 320000
_H1 = 64
_H2 = 128
_NCLS = 15

_NC = 2
_NS = 16
_NW = _NC * _NS
_B = 128
_NB = 80
_EPAD = _NW * _NB * _B
_NPAD = 10240
_RPT = _NPAD // _NS
_XROWS = 10240
_DBLK = _NPAD // 128
_CH = 16


def _mesh():
    return plsc.VectorSubcoreMesh(core_axis_name="c", subcore_axis_name="s")


_SC_PARAMS = pltpu.CompilerParams(use_tc_tiling_on_sc=False)


def _deg_body(eidx, dsrc_out, ddst_out,
              sidx, didx, ones_v, tb, tb2, dsrc_sh, ddst_sh, dsem):
    c = lax.axis_index("c")
    s = lax.axis_index("s")
    w = c * _NS + s
    for i in range(8):
        ones_v[pl.ds(i * 16, 16)] = jnp.ones((16,), jnp.float32)
    for i in range(40):
        tb[pl.ds(i * 16, 16)] = jnp.zeros((16,), jnp.float32)
    pltpu.sync_copy(tb, dsrc_sh.at[pl.ds(s * _RPT, _RPT)])
    pltpu.sync_copy(tb, ddst_sh.at[pl.ds(s * _RPT, _RPT)])
    pltpu.sync_copy(eidx.at[0, w], sidx)
    pltpu.sync_copy(eidx.at[1, w], didx)
    plsc.subcore_barrier()

    @pl.loop(0, _NB)
    def _(j):
        pltpu.async_copy(ones_v, dsrc_sh.at[sidx.at[j]], dsem, add=True)
        pltpu.async_copy(ones_v, ddst_sh.at[didx.at[j]], dsem, add=True)

    @pl.loop(0, 2 * _NB)
    def _(j):
        pltpu.make_async_copy(ones_v, dsrc_sh.at[sidx.at[0]], dsem).wait()

    plsc.subcore_barrier()
    for sh, out in ((dsrc_sh, dsrc_out), (ddst_sh, ddst_out)):
        pltpu.sync_copy(sh.at[pl.ds(s * _RPT, _RPT)], tb)
        for i in range(5):
            for l in range(8):
                tb2[i, pl.ds(l * 16, 16)] = tb[pl.ds(i * 128 + l * 16, 16)]
        pltpu.sync_copy(tb2, out.at[c, pl.ds(s * 5, 5)])


_deg = functools.partial(
    pl.kernel,
    out_type=[jax.ShapeDtypeStruct((_NC, _DBLK, 128), jnp.float32),
              jax.ShapeDtypeStruct((_NC, _DBLK, 128), jnp.float32)],
    mesh=_mesh(),
    compiler_params=_SC_PARAMS,
    scratch_types=[
        pltpu.VMEM((_NB, _B), jnp.int32),
        pltpu.VMEM((_NB, _B), jnp.int32),
        pltpu.VMEM((_B,), jnp.float32),
        pltpu.VMEM((_RPT,), jnp.float32),
        pltpu.VMEM((5, 128), jnp.float32),
        pltpu.VMEM_SHARED((_NPAD,), jnp.float32),
        pltpu.VMEM_SHARED((_NPAD,), jnp.float32),
        pltpu.SemaphoreType.DMA,
    ],
)(_deg_body)


def _make_scatter(nb, colsplit):
    def body(xs, eidx, agg_out, sidx, didx, b0, b1, stg, cstg,
             xs_sh, agg_sh, gs0, gs1, ss0, ss1):
        c = lax.axis_index("c")
        s = lax.axis_index("s")
        g = s if colsplit else c * _NS + s
        coff = c * _H1 if colsplit else 0

        @pl.loop(0, 128)
        def _(r):
            for l in range(4):
                cstg[r, pl.ds(l * 16, 16)] = jnp.zeros((16,), jnp.float32)

        for k in range(5):
            pltpu.sync_copy(cstg, agg_sh.at[pl.ds(s * _RPT + k * 128, 128)])
        pltpu.sync_copy(xs.at[pl.ds(s * 640, 640), pl.ds(coff, _H1)],
                        xs_sh.at[pl.ds(s * 640, 640)])
        plsc.subcore_barrier()

        @pl.loop(0, nb, step=_CH)
        def _(jc):
            pltpu.sync_copy(eidx.at[0, g, pl.ds(jc, _CH)], sidx)
            pltpu.sync_copy(eidx.at[1, g, pl.ds(jc, _CH)], didx)
            pltpu.async_copy(xs_sh.at[sidx.at[0]], b0, gs0)
            pltpu.async_copy(xs_sh.at[sidx.at[1]], b1, gs1)
            for u in range(_CH // 2):
                if u > 0:
                    pltpu.make_async_copy(b0, agg_sh.at[didx.at[0]],
                                          ss0).wait()
                    pltpu.async_copy(xs_sh.at[sidx.at[2 * u]], b0, gs0)
                    pltpu.make_async_copy(b1, agg_sh.at[didx.at[0]],
                                          ss1).wait()
                    pltpu.async_copy(xs_sh.at[sidx.at[2 * u + 1]], b1, gs1)
                pltpu.make_async_copy(xs_sh.at[sidx.at[0]], b0, gs0).wait()
                pltpu.async_copy(b0, agg_sh.at[didx.at[2 * u]], ss0,
                                 add=True)
                pltpu.make_async_copy(xs_sh.at[sidx.at[0]], b1, gs1).wait()
                pltpu.async_copy(b1, agg_sh.at[didx.at[2 * u + 1]], ss1,
                                 add=True)
            pltpu.make_async_copy(b0, agg_sh.at[didx.at[0]], ss0).wait()
            pltpu.make_async_copy(b1, agg_sh.at[didx.at[0]], ss1).wait()

        plsc.subcore_barrier()
        for k in range(5):
            pltpu.sync_copy(agg_sh.at[pl.ds(s * _RPT + k * 128, 128)], cstg)

            @pl.loop(0, 64)
            def _(i):
                for l in range(8):
                    stg[i, pl.ds(l * 16, 16)] = (
                        cstg[2 * i + l // 4, pl.ds((l % 4) * 16, 16)])

            pltpu.sync_copy(stg.at[pl.ds(0, 64)],
                            agg_out.at[c, pl.ds(s * 320 + k * 64, 64)])

    return functools.partial(
        pl.kernel,
        out_type=jax.ShapeDtypeStruct((_NC, _NPAD // 2, 128), jnp.float32),
        mesh=_mesh(),
        compiler_params=_SC_PARAMS,
        scratch_types=[
            pltpu.VMEM((_CH, _B), jnp.int32),
            pltpu.VMEM((_CH, _B), jnp.int32),
            pltpu.VMEM((_B, _H1), jnp.float32),
            pltpu.VMEM((_B, _H1), jnp.float32),
            pltpu.VMEM((128, 128), jnp.float32),
            pltpu.VMEM((128, _H1), jnp.float32),
            pltpu.VMEM_SHARED((_XROWS, _H1), jnp.float32),
            pltpu.VMEM_SHARED((_NPAD, _H1), jnp.float32),
            pltpu.SemaphoreType.DMA,
            pltpu.SemaphoreType.DMA,
            pltpu.SemaphoreType.DMA,
            pltpu.SemaphoreType.DMA,
        ],
    )(body)


_edge_scatter_l1 = _make_scatter(_NB, colsplit=False)
_edge_scatter_l2 = _make_scatter(2 * _NB, colsplit=True)


def _norm(deg_ref):
    d = deg_ref[0] + deg_ref[1]
    return lax.rsqrt(jnp.maximum(d, 1.0))


def _tc1a_body(feat, w1, xw1):
    xw1[...] = jnp.dot(feat[...], w1[...],
                       preferred_element_type=jnp.float32)


def _tc1_body(xw1, dsrc, xs1):
    nsrc = _norm(dsrc)
    xs1[...] = jnp.zeros((_XROWS, 128), jnp.float32)
    xs1[0:_N, 0:_H1] = xw1[...] * nsrc


def _tc2_body(agg, dsrc, ddst, b1, w2, xs2):
    ndst = _norm(ddst)
    nsrc = _norm(dsrc)
    x1 = jax.nn.relu((agg[0] + agg[1]) * ndst + b1[...])
    xs2[...] = jnp.zeros((_XROWS, 128), jnp.float32)
    xs2[0:_N, :] = jnp.dot(x1, w2[...],
                           preferred_element_type=jnp.float32) * nsrc


def _tc3_body(agg, ddst, b2, fw1, fb1, fw2, fb2, out):
    ndst = _norm(ddst)
    x2c = jnp.concatenate([agg[0], agg[1]], axis=-1)
    x2 = jax.nn.relu(x2c * ndst + b2[...])
    m = jnp.sum(x2, axis=0, keepdims=True) * (1.0 / _N)
    h = jax.nn.relu(jnp.dot(m, fw1[...], preferred_element_type=jnp.float32)
                    + fb1[...])
    out[...] = jnp.dot(h, fw2[...], preferred_element_type=jnp.float32) + fb2[...]


def kernel(features, edge_index, W1, b1, W2, b2, fc1_w, fc1_b, fc2_w, fc2_b):
    pad = jnp.full((2, _EPAD - _E), _N, jnp.int32)
    ei = jnp.concatenate([edge_index, pad], axis=1).reshape(2, _NW, _NB, _B)
    ei2 = ei.reshape(2, _NS, 2 * _NB, _B)

    xw1 = pl.pallas_call(
        _tc1a_body,
        out_shape=jax.ShapeDtypeStruct((_N, _H1), jnp.float32),
    )(features, W1)

    dsrc_r, ddst_r = _deg(ei)
    dsrc = dsrc_r.reshape(_NC, _NPAD)[:, :_N].reshape(_NC, _N, 1)
    ddst = ddst_r.reshape(_NC, _NPAD)[:, :_N].reshape(_NC, _N, 1)

    xs1 = pl.pallas_call(
        _tc1_body,
        out_shape=jax.ShapeDtypeStruct((_XROWS, 128), jnp.float32),
    )(xw1, dsrc)

    agg1 = _edge_scatter_l1(xs1, ei).reshape(_NC, _NPAD, _H1)[:, :_N]

    xs2 = pl.pallas_call(
        _tc2_body,
        out_shape=jax.ShapeDtypeStruct((_XROWS, 128), jnp.float32),
    )(agg1, dsrc, ddst, b1, W2)

    agg2 = _edge_scatter_l2(xs2, ei2).reshape(_NC, _NPAD, _H1)[:, :_N]

    out = pl.pallas_call(
        _tc3_body,
        out_shape=jax.ShapeDtypeStruct((1, _NCLS), jnp.float32),
    )(agg2, ddst, b2, fc1_w, fc1_b, fc2_w, fc2_b)

    return out.reshape(_NCLS)


# speedup vs baseline: 1.0059x; 1.0059x over previous
import functools

import jax
import jax.numpy as jnp
from jax import lax
from jax.experimental import pallas as pl
from jax.experimental.pallas import tpu as pltpu
from jax.experimental.pallas import tpu_sc as plsc

_N = 10000
_E = 320000
_H1 = 64
_H2 = 128
_NCLS = 15

_NC = 2
_NS = 16
_NW = _NC * _NS
_B = 128
_NB = 80
_EPAD = _NW * _NB * _B
_NPAD = 10240
_RPT = _NPAD // _NS
_XROWS = 10240
_DBLK = _NPAD // 128
_CH = 16


def _mesh():
    return plsc.VectorSubcoreMesh(core_axis_name="c", subcore_axis_name="s")


_SC_PARAMS = pltpu.CompilerParams(use_tc_tiling_on_sc=False)


def _deg_body(eidx, dsrc_out, ddst_out,
              sidx, didx, ones_v, tb, tb2, dsrc_sh, ddst_sh):
    c = lax.axis_index("c")
    s = lax.axis_index("s")
    w = c * _NS + s
    for i in range(8):
        ones_v[pl.ds(i * 16, 16)] = jnp.ones((16,), jnp.float32)
    for i in range(40):
        tb[pl.ds(i * 16, 16)] = jnp.zeros((16,), jnp.float32)
    pltpu.sync_copy(tb, dsrc_sh.at[pl.ds(s * _RPT, _RPT)])
    pltpu.sync_copy(tb, ddst_sh.at[pl.ds(s * _RPT, _RPT)])
    pltpu.sync_copy(eidx.at[0, w], sidx)
    pltpu.sync_copy(eidx.at[1, w], didx)
    plsc.subcore_barrier()

    @pl.loop(0, _NB)
    def _(j):
        pltpu.sync_copy(ones_v, dsrc_sh.at[sidx.at[j]], add=True)
        pltpu.sync_copy(ones_v, ddst_sh.at[didx.at[j]], add=True)

    plsc.subcore_barrier()
    for sh, out in ((dsrc_sh, dsrc_out), (ddst_sh, ddst_out)):
        pltpu.sync_copy(sh.at[pl.ds(s * _RPT, _RPT)], tb)
        for i in range(5):
            for l in range(8):
                tb2[i, pl.ds(l * 16, 16)] = tb[pl.ds(i * 128 + l * 16, 16)]
        pltpu.sync_copy(tb2, out.at[c, pl.ds(s * 5, 5)])


_deg = functools.partial(
    pl.kernel,
    out_type=[jax.ShapeDtypeStruct((_NC, _DBLK, 128), jnp.float32),
              jax.ShapeDtypeStruct((_NC, _DBLK, 128), jnp.float32)],
    mesh=_mesh(),
    compiler_params=_SC_PARAMS,
    scratch_types=[
        pltpu.VMEM((_NB, _B), jnp.int32),
        pltpu.VMEM((_NB, _B), jnp.int32),
        pltpu.VMEM((_B,), jnp.float32),
        pltpu.VMEM((_RPT,), jnp.float32),
        pltpu.VMEM((5, 128), jnp.float32),
        pltpu.VMEM_SHARED((_NPAD,), jnp.float32),
        pltpu.VMEM_SHARED((_NPAD,), jnp.float32),
    ],
)(_deg_body)


def _make_scatter(nb, colsplit):
    def body(xs, eidx, agg_out, sidx, didx, b0, b1, stg, cstg,
             xs_sh, agg_sh, gs0, gs1, ss0, ss1):
        c = lax.axis_index("c")
        s = lax.axis_index("s")
        g = s if colsplit else c * _NS + s
        coff = c * _H1 if colsplit else 0

        @pl.loop(0, 128)
        def _(r):
            for l in range(4):
                cstg[r, pl.ds(l * 16, 16)] = jnp.zeros((16,), jnp.float32)

        for k in range(5):
            pltpu.sync_copy(cstg, agg_sh.at[pl.ds(s * _RPT + k * 128, 128)])
        pltpu.sync_copy(xs.at[pl.ds(s * 640, 640), pl.ds(coff, _H1)],
                        xs_sh.at[pl.ds(s * 640, 640)])
        plsc.subcore_barrier()

        @pl.loop(0, nb, step=_CH)
        def _(jc):
            pltpu.sync_copy(eidx.at[0, g, pl.ds(jc, _CH)], sidx)
            pltpu.sync_copy(eidx.at[1, g, pl.ds(jc, _CH)], didx)
            pltpu.async_copy(xs_sh.at[sidx.at[0]], b0, gs0)
            pltpu.async_copy(xs_sh.at[sidx.at[1]], b1, gs1)
            for u in range(_CH // 2):
                if u > 0:
                    pltpu.make_async_copy(b0, agg_sh.at[didx.at[0]],
                                          ss0).wait()
                    pltpu.async_copy(xs_sh.at[sidx.at[2 * u]], b0, gs0)
                    pltpu.make_async_copy(b1, agg_sh.at[didx.at[0]],
                                          ss1).wait()
                    pltpu.async_copy(xs_sh.at[sidx.at[2 * u + 1]], b1, gs1)
                pltpu.make_async_copy(xs_sh.at[sidx.at[0]], b0, gs0).wait()
                pltpu.async_copy(b0, agg_sh.at[didx.at[2 * u]], ss0,
                                 add=True)
                pltpu.make_async_copy(xs_sh.at[sidx.at[0]], b1, gs1).wait()
                pltpu.async_copy(b1, agg_sh.at[didx.at[2 * u + 1]], ss1,
                                 add=True)
            pltpu.make_async_copy(b0, agg_sh.at[didx.at[0]], ss0).wait()
            pltpu.make_async_copy(b1, agg_sh.at[didx.at[0]], ss1).wait()

        plsc.subcore_barrier()
        for k in range(5):
            pltpu.sync_copy(agg_sh.at[pl.ds(s * _RPT + k * 128, 128)], cstg)

            @pl.loop(0, 64)
            def _(i):
                for l in range(8):
                    stg[i, pl.ds(l * 16, 16)] = (
                        cstg[2 * i + l // 4, pl.ds((l % 4) * 16, 16)])

            pltpu.sync_copy(stg.at[pl.ds(0, 64)],
                            agg_out.at[c, pl.ds(s * 320 + k * 64, 64)])

    return functools.partial(
        pl.kernel,
        out_type=jax.ShapeDtypeStruct((_NC, _NPAD // 2, 128), jnp.float32),
        mesh=_mesh(),
        compiler_params=_SC_PARAMS,
        scratch_types=[
            pltpu.VMEM((_CH, _B), jnp.int32),
            pltpu.VMEM((_CH, _B), jnp.int32),
            pltpu.VMEM((_B, _H1), jnp.float32),
            pltpu.VMEM((_B, _H1), jnp.float32),
            pltpu.VMEM((128, 128), jnp.float32),
            pltpu.VMEM((128, _H1), jnp.float32),
            pltpu.VMEM_SHARED((_XROWS, _H1), jnp.float32),
            pltpu.VMEM_SHARED((_NPAD, _H1), jnp.float32),
            pltpu.SemaphoreType.DMA,
            pltpu.SemaphoreType.DMA,
            pltpu.SemaphoreType.DMA,
            pltpu.SemaphoreType.DMA,
        ],
    )(body)


_edge_scatter_l1 = _make_scatter(_NB, colsplit=False)
_edge_scatter_l2 = _make_scatter(2 * _NB, colsplit=True)


def _norm(deg_ref):
    d = deg_ref[0] + deg_ref[1]
    return lax.rsqrt(jnp.maximum(d, 1.0))


def _tc1_body(feat, w1, dsrc, xs1):
    nsrc = _norm(dsrc)
    xs1[...] = jnp.zeros((_XROWS, 128), jnp.float32)
    xs1[0:_N, 0:_H1] = jnp.dot(feat[...], w1[...],
                               preferred_element_type=jnp.float32) * nsrc


def _tc2_body(agg, dsrc, ddst, b1, w2, xs2):
    ndst = _norm(ddst)
    nsrc = _norm(dsrc)
    x1 = jax.nn.relu((agg[0] + agg[1]) * ndst + b1[...])
    xs2[...] = jnp.zeros((_XROWS, 128), jnp.float32)
    xs2[0:_N, :] = jnp.dot(x1, w2[...],
                           preferred_element_type=jnp.float32) * nsrc


def _tc3_body(agg, ddst, b2, fw1, fb1, fw2, fb2, out):
    ndst = _norm(ddst)
    x2c = jnp.concatenate([agg[0], agg[1]], axis=-1)
    x2 = jax.nn.relu(x2c * ndst + b2[...])
    m = jnp.sum(x2, axis=0, keepdims=True) * (1.0 / _N)
    h = jax.nn.relu(jnp.dot(m, fw1[...], preferred_element_type=jnp.float32)
                    + fb1[...])
    out[...] = jnp.dot(h, fw2[...], preferred_element_type=jnp.float32) + fb2[...]


def kernel(features, edge_index, W1, b1, W2, b2, fc1_w, fc1_b, fc2_w, fc2_b):
    pad = jnp.full((2, _EPAD - _E), _N, jnp.int32)
    ei = jnp.concatenate([edge_index, pad], axis=1).reshape(2, _NW, _NB, _B)
    ei2 = ei.reshape(2, _NS, 2 * _NB, _B)

    dsrc_r, ddst_r = _deg(ei)
    dsrc = dsrc_r.reshape(_NC, _NPAD)[:, :_N].reshape(_NC, _N, 1)
    ddst = ddst_r.reshape(_NC, _NPAD)[:, :_N].reshape(_NC, _N, 1)

    xs1 = pl.pallas_call(
        _tc1_body,
        out_shape=jax.ShapeDtypeStruct((_XROWS, 128), jnp.float32),
    )(features, W1, dsrc)

    agg1 = _edge_scatter_l1(xs1, ei).reshape(_NC, _NPAD, _H1)[:, :_N]

    xs2 = pl.pallas_call(
        _tc2_body,
        out_shape=jax.ShapeDtypeStruct((_XROWS, 128), jnp.float32),
    )(agg1, dsrc, ddst, b1, W2)

    agg2 = _edge_scatter_l2(xs2, ei2).reshape(_NC, _NPAD, _H1)[:, :_N]

    out = pl.pallas_call(
        _tc3_body,
        out_shape=jax.ShapeDtypeStruct((1, _NCLS), jnp.float32),
    )(agg2, ddst, b2, fc1_w, fc1_b, fc2_w, fc2_b)

    return out.reshape(_NCLS)
